# baseline (speedup 1.0000x reference)
import functools

import jax
import jax.numpy as jnp
from jax import lax
from jax.experimental import pallas as pl
from jax.experimental.pallas import tpu as pltpu
from jax.experimental.pallas import tpu_sc as plsc

NUM_CORES = 2
NUM_SUBCORES = 16
LANES = 16
NW = NUM_CORES * NUM_SUBCORES

B_PER_W = 512
CHUNK = 128
NCHUNK = B_PER_W // CHUNK
NG = CHUNK // LANES
DCAT = 128


def _mf_body(uidx_hbm, iidx_hbm, tab_hbm, out_hbm,
             idx_r, idx_s, data_u, data_i, out_v, sem_a, sem_b):
    d_model = DCAT // 2
    wid = lax.axis_index("s") * NUM_CORES + lax.axis_index("c")
    base = wid * B_PER_W

    pltpu.sync_copy(uidx_hbm.at[pl.ds(base, B_PER_W)], idx_r)
    pltpu.sync_copy(iidx_hbm.at[pl.ds(base, B_PER_W)], idx_s)

    iota16 = lax.iota(jnp.int32, LANES)
    sems = (sem_a, sem_b)

    def fire(c):
        p = c % 2
        pltpu.async_copy(tab_hbm.at[idx_r.at[pl.ds(c * CHUNK, CHUNK)]],
                         data_u.at[p], sems[p])
        pltpu.async_copy(tab_hbm.at[idx_s.at[pl.ds(c * CHUNK, CHUNK)]],
                         data_i.at[p], sems[p])

    fire(0)
    for c in range(NCHUNK):
        p = c % 2
        if c + 1 < NCHUNK:
            fire(c + 1)
        pltpu.make_async_copy(tab_hbm.at[pl.ds(0, CHUNK)],
                              data_u.at[p], sems[p]).wait()
        pltpu.make_async_copy(tab_hbm.at[pl.ds(0, CHUNK)],
                              data_i.at[p], sems[p]).wait()

        pv = jnp.full((LANES,), p, jnp.int32)

        def gbody(g, carry, pv=pv, c=c):
            row16 = g * LANES + iota16
            acc = jnp.zeros((LANES,), jnp.float32)
            for d in range(d_model):
                dv = jnp.full((LANES,), d, jnp.int32)
                u = plsc.load_gather(data_u, [pv, row16, dv])
                w = plsc.load_gather(data_i, [pv, row16, dv + d_model])
                acc = acc + u * w
            out_v[pl.ds(c * CHUNK + g * LANES, LANES)] = acc
            return carry

        lax.fori_loop(0, NG, gbody, 0)

    pltpu.sync_copy(out_v, out_hbm.at[pl.ds(base, B_PER_W)])


def _relayout_body(u_ref, i_ref, out_ref):
    cat = jnp.concatenate([u_ref[...], i_ref[...]], axis=0)
    out_ref[...] = cat.T


def _build_table(user_table, item_table):
    v_rows, d_model = user_table.shape
    cw = 512
    nblk = (v_rows + cw - 1) // cw
    return pl.pallas_call(
        _relayout_body,
        grid=(nblk,),
        in_specs=[
            pl.BlockSpec((d_model, cw), lambda j: (0, j)),
            pl.BlockSpec((d_model, cw), lambda j: (0, j)),
        ],
        out_specs=pl.BlockSpec((cw, DCAT), lambda j: (j, 0)),
        out_shape=jax.ShapeDtypeStruct((v_rows, DCAT), jnp.float32),
    )(user_table.T, item_table.T)


@jax.jit
def kernel(user_idx, item_idx, user_table, item_table):
    batch = user_idx.shape[0]
    v_rows, d_model = user_table.shape
    assert batch == NW * B_PER_W, batch
    assert 2 * d_model == DCAT

    tab = _build_table(user_table, item_table)
    uidx = user_idx.astype(jnp.int32)
    iidx = item_idx.astype(jnp.int32)

    mesh = plsc.VectorSubcoreMesh(core_axis_name="c", subcore_axis_name="s",
                                  num_cores=NUM_CORES,
                                  num_subcores=NUM_SUBCORES)
    kfn = pl.kernel(
        _mf_body,
        out_type=jax.ShapeDtypeStruct((batch,), jnp.float32),
        mesh=mesh,
        compiler_params=pltpu.CompilerParams(needs_layout_passes=False,
                                             use_tc_tiling_on_sc=False),
        scratch_types=[
            pltpu.VMEM((B_PER_W,), jnp.int32),
            pltpu.VMEM((B_PER_W,), jnp.int32),
            pltpu.VMEM((2, CHUNK, DCAT), jnp.float32),
            pltpu.VMEM((2, CHUNK, DCAT), jnp.float32),
            pltpu.VMEM((B_PER_W,), jnp.float32),
            pltpu.SemaphoreType.DMA,
            pltpu.SemaphoreType.DMA,
        ],
    )
    return kfn(uidx, iidx, tab)


# speedup vs baseline: 2.2477x; 2.2477x over previous
import functools

import jax
import jax.numpy as jnp
from jax import lax
from jax.experimental import pallas as pl
from jax.experimental.pallas import tpu as pltpu
from jax.experimental.pallas import tpu_sc as plsc

NUM_CORES = 2
NUM_SUBCORES = 16
LANES = 16
NW = NUM_CORES * NUM_SUBCORES

B_PER_W = 512
CHUNK = 128
NCHUNK = B_PER_W // CHUNK
NG = CHUNK // LANES
DCAT = 128


def _mf_body(uidx_hbm, iidx_hbm, tab_hbm, out_hbm,
             idx_r, idx_s, data_u, data_i, out_v, sem_a, sem_b):
    d_model = DCAT // 2
    wid = lax.axis_index("s") * NUM_CORES + lax.axis_index("c")
    base = wid * B_PER_W

    pltpu.sync_copy(uidx_hbm.at[pl.ds(base, B_PER_W)], idx_r)
    pltpu.sync_copy(iidx_hbm.at[pl.ds(base, B_PER_W)], idx_s)

    iota16 = lax.iota(jnp.int32, LANES)
    sems = (sem_a, sem_b)

    def fire(c):
        p = c % 2
        pltpu.async_copy(tab_hbm.at[idx_r.at[pl.ds(c * CHUNK, CHUNK)]],
                         data_u.at[p], sems[p])
        pltpu.async_copy(tab_hbm.at[idx_s.at[pl.ds(c * CHUNK, CHUNK)]],
                         data_i.at[p], sems[p])

    fire(0)
    for c in range(NCHUNK):
        p = c % 2
        if c + 1 < NCHUNK:
            fire(c + 1)
        pltpu.make_async_copy(tab_hbm.at[pl.ds(0, CHUNK)],
                              data_u.at[p], sems[p]).wait()
        pltpu.make_async_copy(tab_hbm.at[pl.ds(0, CHUNK)],
                              data_i.at[p], sems[p]).wait()

        pv = jnp.full((LANES,), p, jnp.int32)

        def gbody(g, carry, pv=pv, c=c):
            row16 = g * LANES + iota16
            acc = jnp.zeros((LANES,), jnp.float32)
            for d in range(d_model):
                dv = jnp.full((LANES,), d, jnp.int32)
                u = plsc.load_gather(data_u, [pv, row16, dv])
                w = plsc.load_gather(data_i, [pv, row16, dv + d_model])
                acc = acc + u * w
            out_v[pl.ds(c * CHUNK + g * LANES, LANES)] = acc
            return carry

        lax.fori_loop(0, NG, gbody, 0)

    pltpu.sync_copy(out_v, out_hbm.at[pl.ds(base, B_PER_W)])


def _relayout_body(u_ref, i_ref, out_ref):
    cat = jnp.concatenate([u_ref[...], i_ref[...]], axis=0)
    eye = jnp.eye(DCAT, dtype=jnp.float32)
    out_ref[...] = jax.lax.dot_general(
        cat, eye, (((0,), (0,)), ((), ())),
        preferred_element_type=jnp.float32)


def _build_table(user_table, item_table):
    v_rows, d_model = user_table.shape
    cw = 2048
    nblk = (v_rows + cw - 1) // cw
    return pl.pallas_call(
        _relayout_body,
        grid=(nblk,),
        in_specs=[
            pl.BlockSpec((d_model, cw), lambda j: (0, j)),
            pl.BlockSpec((d_model, cw), lambda j: (0, j)),
        ],
        out_specs=pl.BlockSpec((cw, DCAT), lambda j: (j, 0)),
        out_shape=jax.ShapeDtypeStruct((v_rows, DCAT), jnp.float32),
    )(user_table.T, item_table.T)


@jax.jit
def kernel(user_idx, item_idx, user_table, item_table):
    batch = user_idx.shape[0]
    v_rows, d_model = user_table.shape
    assert batch == NW * B_PER_W, batch
    assert 2 * d_model == DCAT

    tab = _build_table(user_table, item_table)
    uidx = user_idx.astype(jnp.int32)
    iidx = item_idx.astype(jnp.int32)

    mesh = plsc.VectorSubcoreMesh(core_axis_name="c", subcore_axis_name="s",
                                  num_cores=NUM_CORES,
                                  num_subcores=NUM_SUBCORES)
    kfn = pl.kernel(
        _mf_body,
        out_type=jax.ShapeDtypeStruct((batch,), jnp.float32),
        mesh=mesh,
        compiler_params=pltpu.CompilerParams(needs_layout_passes=False,
                                             use_tc_tiling_on_sc=False),
        scratch_types=[
            pltpu.VMEM((B_PER_W,), jnp.int32),
            pltpu.VMEM((B_PER_W,), jnp.int32),
            pltpu.VMEM((2, CHUNK, DCAT), jnp.float32),
            pltpu.VMEM((2, CHUNK, DCAT), jnp.float32),
            pltpu.VMEM((B_PER_W,), jnp.float32),
            pltpu.SemaphoreType.DMA,
            pltpu.SemaphoreType.DMA,
        ],
    )
    return kfn(uidx, iidx, tab)


# speedup vs baseline: 2.2556x; 1.0035x over previous
import functools

import jax
import jax.numpy as jnp
from jax import lax
from jax.experimental import pallas as pl
from jax.experimental.pallas import tpu as pltpu
from jax.experimental.pallas import tpu_sc as plsc

NUM_CORES = 2
NUM_SUBCORES = 16
LANES = 16
NW = NUM_CORES * NUM_SUBCORES

B_PER_W = 512
CHUNK = 128
NCHUNK = B_PER_W // CHUNK
NG = CHUNK // LANES
DCAT = 128


def _mf_body(uidx_hbm, iidx_hbm, tab_hbm, out_hbm,
             idx_r, idx_s, data_u, data_i, out_v, sem_a, sem_b):
    d_model = DCAT // 2
    wid = lax.axis_index("s") * NUM_CORES + lax.axis_index("c")
    base = wid * B_PER_W

    pltpu.sync_copy(uidx_hbm.at[pl.ds(base, B_PER_W)], idx_r)
    pltpu.sync_copy(iidx_hbm.at[pl.ds(base, B_PER_W)], idx_s)

    iota16 = lax.iota(jnp.int32, LANES)
    sems = (sem_a, sem_b)

    def fire(c):
        p = c % 2
        pltpu.async_copy(tab_hbm.at[idx_r.at[pl.ds(c * CHUNK, CHUNK)]],
                         data_u.at[p], sems[p])
        pltpu.async_copy(tab_hbm.at[idx_s.at[pl.ds(c * CHUNK, CHUNK)]],
                         data_i.at[p], sems[p])

    fire(0)
    for c in range(NCHUNK):
        p = c % 2
        if c + 1 < NCHUNK:
            fire(c + 1)
        pltpu.make_async_copy(tab_hbm.at[pl.ds(0, CHUNK)],
                              data_u.at[p], sems[p]).wait()
        pltpu.make_async_copy(tab_hbm.at[pl.ds(0, CHUNK)],
                              data_i.at[p], sems[p]).wait()

        pv = jnp.full((LANES,), p, jnp.int32)

        def gbody(g, carry, pv=pv, c=c):
            row16 = g * LANES + iota16
            acc = jnp.zeros((LANES,), jnp.float32)
            for d in range(d_model):
                dv = jnp.full((LANES,), d, jnp.int32)
                u = plsc.load_gather(data_u, [pv, row16, dv])
                w = plsc.load_gather(data_i, [pv, row16, dv + d_model])
                acc = acc + u * w
            out_v[pl.ds(c * CHUNK + g * LANES, LANES)] = acc
            return carry

        lax.fori_loop(0, NG, gbody, 0)

    pltpu.sync_copy(out_v, out_hbm.at[pl.ds(base, B_PER_W)])


def _relayout_body(u_ref, i_ref, out_ref):
    cat = jnp.concatenate([u_ref[...], i_ref[...]], axis=0)
    eye = jnp.eye(DCAT, dtype=jnp.float32)
    out_ref[...] = jax.lax.dot_general(
        cat, eye, (((0,), (0,)), ((), ())),
        precision=jax.lax.Precision.HIGHEST,
        preferred_element_type=jnp.float32)


def _build_table(user_table, item_table):
    v_rows, d_model = user_table.shape
    cw = 4096
    nblk = (v_rows + cw - 1) // cw
    return pl.pallas_call(
        _relayout_body,
        grid=(nblk,),
        in_specs=[
            pl.BlockSpec((d_model, cw), lambda j: (0, j)),
            pl.BlockSpec((d_model, cw), lambda j: (0, j)),
        ],
        out_specs=pl.BlockSpec((cw, DCAT), lambda j: (j, 0)),
        out_shape=jax.ShapeDtypeStruct((v_rows, DCAT), jnp.float32),
    )(user_table.T, item_table.T)


@jax.jit
def kernel(user_idx, item_idx, user_table, item_table):
    batch = user_idx.shape[0]
    v_rows, d_model = user_table.shape
    assert batch == NW * B_PER_W, batch
    assert 2 * d_model == DCAT

    tab = _build_table(user_table, item_table)
    uidx = user_idx.astype(jnp.int32)
    iidx = item_idx.astype(jnp.int32)

    mesh = plsc.VectorSubcoreMesh(core_axis_name="c", subcore_axis_name="s",
                                  num_cores=NUM_CORES,
                                  num_subcores=NUM_SUBCORES)
    kfn = pl.kernel(
        _mf_body,
        out_type=jax.ShapeDtypeStruct((batch,), jnp.float32),
        mesh=mesh,
        compiler_params=pltpu.CompilerParams(needs_layout_passes=False,
                                             use_tc_tiling_on_sc=False),
        scratch_types=[
            pltpu.VMEM((B_PER_W,), jnp.int32),
            pltpu.VMEM((B_PER_W,), jnp.int32),
            pltpu.VMEM((2, CHUNK, DCAT), jnp.float32),
            pltpu.VMEM((2, CHUNK, DCAT), jnp.float32),
            pltpu.VMEM((B_PER_W,), jnp.float32),
            pltpu.SemaphoreType.DMA,
            pltpu.SemaphoreType.DMA,
        ],
    )
    return kfn(uidx, iidx, tab)


# speedup vs baseline: 3.0124x; 1.3355x over previous
import functools

import jax
import jax.numpy as jnp
from jax import lax
from jax.experimental import pallas as pl
from jax.experimental.pallas import tpu as pltpu
from jax.experimental.pallas import tpu_sc as plsc

NUM_CORES = 2
NUM_SUBCORES = 16
LANES = 16
NW = NUM_CORES * NUM_SUBCORES

B_PER_W = 512
CHUNK = 128
NCHUNK = B_PER_W // CHUNK
NG = CHUNK // LANES
DCAT = 128


def _mf_body(uidx_hbm, iidx_hbm, tab_hbm, out_hbm,
             idx_r, idx_s, data_u, data_i, out_v, sem_a, sem_b):
    d_model = DCAT // 2
    wid = lax.axis_index("s") * NUM_CORES + lax.axis_index("c")
    base = wid * B_PER_W

    pltpu.sync_copy(uidx_hbm.at[pl.ds(base, B_PER_W)], idx_r)
    pltpu.sync_copy(iidx_hbm.at[pl.ds(base, B_PER_W)], idx_s)

    iota16 = lax.iota(jnp.int32, LANES)
    sems = (sem_a, sem_b)

    def fire(c):
        p = c % 2
        pltpu.async_copy(tab_hbm.at[idx_r.at[pl.ds(c * CHUNK, CHUNK)]],
                         data_u.at[p], sems[p])
        pltpu.async_copy(tab_hbm.at[idx_s.at[pl.ds(c * CHUNK, CHUNK)]],
                         data_i.at[p], sems[p])

    fire(0)
    for c in range(NCHUNK):
        p = c % 2
        if c + 1 < NCHUNK:
            fire(c + 1)
        pltpu.make_async_copy(tab_hbm.at[pl.ds(0, CHUNK)],
                              data_u.at[p], sems[p]).wait()
        pltpu.make_async_copy(tab_hbm.at[pl.ds(0, CHUNK)],
                              data_i.at[p], sems[p]).wait()

        pv = jnp.full((LANES,), p, jnp.int32)

        def gbody(g, carry, pv=pv, c=c):
            row16 = g * LANES + iota16
            acc = jnp.zeros((LANES,), jnp.float32)
            for d in range(d_model):
                dv = jnp.full((LANES,), d, jnp.int32)
                u = plsc.load_gather(data_u, [pv, row16, dv])
                w = plsc.load_gather(data_i, [pv, row16, dv + d_model])
                acc = acc + u * w
            out_v[pl.ds(c * CHUNK + g * LANES, LANES)] = acc
            return carry

        lax.fori_loop(0, NG, gbody, 0)

    pltpu.sync_copy(out_v, out_hbm.at[pl.ds(base, B_PER_W)])


def _relayout_body(u_ref, i_ref, out_ref):
    cat = jnp.concatenate([u_ref[...], i_ref[...]], axis=0)
    out_ref[...] = cat.T


def _build_table(user_table, item_table):
    v_rows, d_model = user_table.shape
    cw = 4096
    nblk = (v_rows + cw - 1) // cw
    return pl.pallas_call(
        _relayout_body,
        grid=(nblk,),
        in_specs=[
            pl.BlockSpec((d_model, cw), lambda j: (0, j)),
            pl.BlockSpec((d_model, cw), lambda j: (0, j)),
        ],
        out_specs=pl.BlockSpec((cw, DCAT), lambda j: (j, 0)),
        out_shape=jax.ShapeDtypeStruct((v_rows, DCAT), jnp.float32),
    )(user_table.T, item_table.T)


@jax.jit
def kernel(user_idx, item_idx, user_table, item_table):
    batch = user_idx.shape[0]
    v_rows, d_model = user_table.shape
    assert batch == NW * B_PER_W, batch
    assert 2 * d_model == DCAT

    tab = _build_table(user_table, item_table)
    uidx = user_idx.astype(jnp.int32)
    iidx = item_idx.astype(jnp.int32)

    mesh = plsc.VectorSubcoreMesh(core_axis_name="c", subcore_axis_name="s",
                                  num_cores=NUM_CORES,
                                  num_subcores=NUM_SUBCORES)
    kfn = pl.kernel(
        _mf_body,
        out_type=jax.ShapeDtypeStruct((batch,), jnp.float32),
        mesh=mesh,
        compiler_params=pltpu.CompilerParams(needs_layout_passes=False,
                                             use_tc_tiling_on_sc=False),
        scratch_types=[
            pltpu.VMEM((B_PER_W,), jnp.int32),
            pltpu.VMEM((B_PER_W,), jnp.int32),
            pltpu.VMEM((2, CHUNK, DCAT), jnp.float32),
            pltpu.VMEM((2, CHUNK, DCAT), jnp.float32),
            pltpu.VMEM((B_PER_W,), jnp.float32),
            pltpu.SemaphoreType.DMA,
            pltpu.SemaphoreType.DMA,
        ],
    )
    return kfn(uidx, iidx, tab)


# speedup vs baseline: 3.4549x; 1.1469x over previous
import functools

import jax
import jax.numpy as jnp
from jax import lax
from jax.experimental import pallas as pl
from jax.experimental.pallas import tpu as pltpu
from jax.experimental.pallas import tpu_sc as plsc

NUM_CORES = 2
NUM_SUBCORES = 16
LANES = 16
NW = NUM_CORES * NUM_SUBCORES

B_PER_W = 512
CHUNK = 128
NCHUNK = B_PER_W // CHUNK
NG = CHUNK // LANES
DCAT = 128


def _mf_body(uidx_hbm, iidx_hbm, tab_hbm, out_hbm,
             idx_r, idx_s, data_u, data_i, out_v, sem_a, sem_b):
    d_model = DCAT // 2
    wid = lax.axis_index("s") * NUM_CORES + lax.axis_index("c")
    base = wid * B_PER_W

    pltpu.sync_copy(uidx_hbm.at[pl.ds(base, B_PER_W)], idx_r)
    pltpu.sync_copy(iidx_hbm.at[pl.ds(base, B_PER_W)], idx_s)

    iota16 = lax.iota(jnp.int32, LANES)
    sems = (sem_a, sem_b)

    def fire(c):
        p = c % 2
        pltpu.async_copy(tab_hbm.at[idx_r.at[pl.ds(c * CHUNK, CHUNK)]],
                         data_u.at[p], sems[p])
        pltpu.async_copy(tab_hbm.at[idx_s.at[pl.ds(c * CHUNK, CHUNK)]],
                         data_i.at[p], sems[p])

    fire(0)
    for c in range(NCHUNK):
        p = c % 2
        if c + 1 < NCHUNK:
            fire(c + 1)
        pltpu.make_async_copy(tab_hbm.at[pl.ds(0, CHUNK)],
                              data_u.at[p], sems[p]).wait()
        pltpu.make_async_copy(tab_hbm.at[pl.ds(0, CHUNK)],
                              data_i.at[p], sems[p]).wait()

        pv = jnp.full((LANES,), p, jnp.int32)

        def gbody(g, carry, pv=pv, c=c):
            row16 = g * LANES + iota16
            acc = jnp.zeros((LANES,), jnp.float32)
            for d in range(d_model):
                dv = jnp.full((LANES,), d, jnp.int32)
                u = plsc.load_gather(data_u, [pv, row16, dv])
                w = plsc.load_gather(data_i, [pv, row16, dv + d_model])
                acc = acc + u * w
            out_v[pl.ds(c * CHUNK + g * LANES, LANES)] = acc
            return carry

        lax.fori_loop(0, NG, gbody, 0)

    pltpu.sync_copy(out_v, out_hbm.at[pl.ds(base, B_PER_W)])


def _relayout_body(u_ref, i_ref, out_ref):
    cat = jnp.concatenate([u_ref[...], i_ref[...]], axis=0)
    out_ref[...] = cat.T


def _build_table(user_table, item_table):
    v_rows, d_model = user_table.shape
    cw = 8192
    nblk = (v_rows + cw - 1) // cw
    return pl.pallas_call(
        _relayout_body,
        grid=(nblk,),
        in_specs=[
            pl.BlockSpec((d_model, cw), lambda j: (0, j)),
            pl.BlockSpec((d_model, cw), lambda j: (0, j)),
        ],
        out_specs=pl.BlockSpec((cw, DCAT), lambda j: (j, 0)),
        out_shape=jax.ShapeDtypeStruct((v_rows, DCAT), jnp.float32),
    )(user_table.T, item_table.T)


@jax.jit
def kernel(user_idx, item_idx, user_table, item_table):
    batch = user_idx.shape[0]
    v_rows, d_model = user_table.shape
    assert batch == NW * B_PER_W, batch
    assert 2 * d_model == DCAT

    tab = _build_table(user_table, item_table)
    uidx = user_idx.astype(jnp.int32)
    iidx = item_idx.astype(jnp.int32)

    mesh = plsc.VectorSubcoreMesh(core_axis_name="c", subcore_axis_name="s",
                                  num_cores=NUM_CORES,
                                  num_subcores=NUM_SUBCORES)
    kfn = pl.kernel(
        _mf_body,
        out_type=jax.ShapeDtypeStruct((batch,), jnp.float32),
        mesh=mesh,
        compiler_params=pltpu.CompilerParams(needs_layout_passes=False,
                                             use_tc_tiling_on_sc=False),
        scratch_types=[
            pltpu.VMEM((B_PER_W,), jnp.int32),
            pltpu.VMEM((B_PER_W,), jnp.int32),
            pltpu.VMEM((2, CHUNK, DCAT), jnp.float32),
            pltpu.VMEM((2, CHUNK, DCAT), jnp.float32),
            pltpu.VMEM((B_PER_W,), jnp.float32),
            pltpu.SemaphoreType.DMA,
            pltpu.SemaphoreType.DMA,
        ],
    )
    return kfn(uidx, iidx, tab)
